# baseline (speedup 1.0000x reference)
import functools

import jax
import jax.numpy as jnp
from jax import lax
from jax.experimental import pallas as pl
from jax.experimental.pallas import tpu as pltpu
from jax.experimental.pallas import tpu_sc as plsc

_NUM_FIELDS = 26
_FIELD_DIM = 40000
_BUCKET = 100000
_D = 64
_GK = 0.02
_QPR = 11
_B = 4096
_N = _B * _NUM_FIELDS


_TC_BLK = 1000


def _prune_body(qv_ref, qs_ref, rv_ref, rs_ref, qd_ref, rd_ref):
    for v_ref, s_ref, o_ref in ((qv_ref, qs_ref, qd_ref),
                                (rv_ref, rs_ref, rd_ref)):
        v = v_ref[...]
        t = _GK * jax.nn.sigmoid(s_ref[...])
        p = v - jnp.minimum(jnp.maximum(v, -t), t)
        a = p[:, :_D]
        b = p[:, _D:]
        o_ref[...] = jnp.concatenate(
            [jnp.concatenate([a, a], axis=1),
             jnp.concatenate([b, b], axis=1)], axis=0)


_prune_tables = pl.pallas_call(
    _prune_body,
    grid=(_BUCKET // (2 * _TC_BLK),),
    in_specs=[pl.BlockSpec((_TC_BLK, 2 * _D), lambda i: (i, 0))] * 4,
    out_specs=[pl.BlockSpec((2 * _TC_BLK, 2 * _D), lambda i: (i, 0))] * 2,
    out_shape=[jax.ShapeDtypeStruct((_BUCKET, 2 * _D), jnp.float32)] * 2,
)


_NC = 2
_NS = 16
_NW = _NC * _NS
_L = 16
_BATCH_PER_W = _B // _NW
_ROWS_PER_W = _N // _NW
_NB = 4
_CHUNK = _NB * _NUM_FIELDS
_NCHUNKS = _BATCH_PER_W // _NB


def _remap(q):
    m = lax.shift_right_logical(q, 1)
    blk = lax.div(m, _TC_BLK)
    within = lax.rem(m, _TC_BLK)
    odd = lax.bitwise_and(q, 1)
    return blk * (2 * _TC_BLK) + odd * _TC_BLK + within


def _sc_body(x_hbm, qd_hbm, rd_hbm, out_hbm,
             xall_v, idxq_v, idxr_v,
             qc_a, rc_a, o_a, qc_b, rc_b, o_b,
             sem_a, sem_b, sem_oa, sem_ob):
    wid = lax.axis_index("s") * _NC + lax.axis_index("c")
    base = wid * _ROWS_PER_W
    bbase = wid * _BATCH_PER_W

    pltpu.sync_copy(x_hbm.at[pl.ds(base, _ROWS_PER_W)], xall_v)
    lane = lax.iota(jnp.int32, _L)

    def idx_body(j, carry):
        sl = pl.ds(j * _L, _L)
        xv = xall_v[sl]
        col = lax.rem(base + j * _L + lane, _NUM_FIELDS)
        xn = xv + col * _FIELD_DIM
        idxq_v[sl] = _remap(lax.div(xn, _QPR))
        idxr_v[sl] = _remap(lax.rem(xn, _BUCKET))
        return carry

    lax.fori_loop(0, _ROWS_PER_W // _L, idx_body, 0, unroll=4)

    def fire_gather(c, qc_v, rc_v, sem):
        iq = idxq_v.at[pl.ds(c * _CHUNK, _CHUNK)]
        ir = idxr_v.at[pl.ds(c * _CHUNK, _CHUNK)]
        pltpu.async_copy(qd_hbm.at[iq], qc_v, sem)
        pltpu.async_copy(rd_hbm.at[ir], rc_v, sem)

    def wait_gather(c, qc_v, rc_v, sem):
        iq = idxq_v.at[pl.ds(c * _CHUNK, _CHUNK)]
        ir = idxr_v.at[pl.ds(c * _CHUNK, _CHUNK)]
        pltpu.make_async_copy(qd_hbm.at[iq], qc_v, sem).wait()
        pltpu.make_async_copy(rd_hbm.at[ir], rc_v, sem).wait()

    def compute(qc_v, rc_v, o_v):
        def b_body(b, carry):
            def f_body(f, carry2):
                i = b * _NUM_FIELDS + f
                for k in range(_D // _L):
                    sl = pl.ds(k * _L, _L)
                    o_v[b, f, sl] = qc_v[i, sl] + rc_v[i, sl]
                return carry2

            lax.fori_loop(0, _NUM_FIELDS, f_body, 0, unroll=2)
            return carry

        lax.fori_loop(0, _NB, b_body, 0)

    def store_slice(c, o_v, sem):
        dst = out_hbm.at[pl.ds(bbase + c * _NB, _NB)]
        return pltpu.make_async_copy(o_v, dst, sem)

    fire_gather(0, qc_a, rc_a, sem_a)
    fire_gather(1, qc_b, rc_b, sem_b)

    def step(g, carry):
        c_a = 2 * g
        c_b = c_a + 1

        wait_gather(c_a, qc_a, rc_a, sem_a)
        compute(qc_a, rc_a, o_a)

        @pl.when(g > 0)
        def _():
            store_slice(c_a - 2, o_a, sem_oa).wait()

        store_slice(c_a, o_a, sem_oa).start()

        @pl.when(g < (_NCHUNKS // 2 - 1))
        def _():
            fire_gather(c_a + 2, qc_a, rc_a, sem_a)

        wait_gather(c_b, qc_b, rc_b, sem_b)
        compute(qc_b, rc_b, o_b)

        @pl.when(g > 0)
        def _():
            store_slice(c_b - 2, o_b, sem_ob).wait()

        store_slice(c_b, o_b, sem_ob).start()

        @pl.when(g < (_NCHUNKS // 2 - 1))
        def _():
            fire_gather(c_b + 2, qc_b, rc_b, sem_b)

        return carry

    lax.fori_loop(0, _NCHUNKS // 2, step, 0)

    store_slice(_NCHUNKS - 2, o_a, sem_oa).wait()
    store_slice(_NCHUNKS - 1, o_b, sem_ob).wait()


_mesh = plsc.VectorSubcoreMesh(core_axis_name="c", subcore_axis_name="s")

_ce_kernel = functools.partial(
    pl.kernel,
    out_type=jax.ShapeDtypeStruct((_B, _NUM_FIELDS, _D), jnp.float32),
    mesh=_mesh,
    scratch_types=[
        pltpu.VMEM((_ROWS_PER_W,), jnp.int32),
        pltpu.VMEM((_ROWS_PER_W,), jnp.int32),
        pltpu.VMEM((_ROWS_PER_W,), jnp.int32),
        pltpu.VMEM((_CHUNK, 2 * _D), jnp.float32),
        pltpu.VMEM((_CHUNK, 2 * _D), jnp.float32),
        pltpu.VMEM((_NB, _NUM_FIELDS, _D), jnp.float32),
        pltpu.VMEM((_CHUNK, 2 * _D), jnp.float32),
        pltpu.VMEM((_CHUNK, 2 * _D), jnp.float32),
        pltpu.VMEM((_NB, _NUM_FIELDS, _D), jnp.float32),
        pltpu.SemaphoreType.DMA,
        pltpu.SemaphoreType.DMA,
        pltpu.SemaphoreType.DMA,
        pltpu.SemaphoreType.DMA,
    ],
    compiler_params=pltpu.CompilerParams(use_tc_tiling_on_sc=True),
)(_sc_body)


def kernel(x, Q_v, R_v, Q_s, R_s):
    half = _BUCKET // 2
    qd, rd = _prune_tables(Q_v.reshape(half, 2 * _D),
                           Q_s.reshape(half, 2 * _D),
                           R_v.reshape(half, 2 * _D),
                           R_s.reshape(half, 2 * _D))
    x_flat = x.reshape(_N)
    return _ce_kernel(x_flat, qd, rd)


# speedup vs baseline: 1.1191x; 1.1191x over previous
import functools

import jax
import jax.numpy as jnp
from jax import lax
from jax.experimental import pallas as pl
from jax.experimental.pallas import tpu as pltpu
from jax.experimental.pallas import tpu_sc as plsc

_NUM_FIELDS = 26
_FIELD_DIM = 40000
_BUCKET = 100000
_D = 64
_GK = 0.02
_QPR = 11
_B = 4096
_N = _B * _NUM_FIELDS


_TC_BLK = 2000


def _prune_body(qv_ref, qs_ref, rv_ref, rs_ref, qd_ref, rd_ref):
    for v_ref, s_ref, o_ref in ((qv_ref, qs_ref, qd_ref),
                                (rv_ref, rs_ref, rd_ref)):
        v = v_ref[...]
        t = _GK * jax.nn.sigmoid(s_ref[...])
        p = v - jnp.minimum(jnp.maximum(v, -t), t)
        o_ref[...] = jnp.concatenate([p, p], axis=1)


_prune_tables = pl.pallas_call(
    _prune_body,
    grid=(_BUCKET // _TC_BLK,),
    in_specs=[pl.BlockSpec((_TC_BLK, _D), lambda i: (i, 0))] * 4,
    out_specs=[pl.BlockSpec((_TC_BLK, 2 * _D), lambda i: (i, 0))] * 2,
    out_shape=[jax.ShapeDtypeStruct((_BUCKET, 2 * _D), jnp.float32)] * 2,
)


_NC = 2
_NS = 16
_NW = _NC * _NS
_L = 16
_BATCH_PER_W = _B // _NW
_ROWS_PER_W = _N // _NW
_NB = 4
_CHUNK = _NB * _NUM_FIELDS
_NCHUNKS = _BATCH_PER_W // _NB


def _sc_body(x_hbm, qd_hbm, rd_hbm, out_hbm,
             xall_v, idxq_v, idxr_v,
             qc_a, rc_a, o_a, qc_b, rc_b, o_b,
             sem_a, sem_b, sem_oa, sem_ob):
    wid = lax.axis_index("s") * _NC + lax.axis_index("c")
    base = wid * _ROWS_PER_W
    bbase = wid * _BATCH_PER_W

    pltpu.sync_copy(x_hbm.at[pl.ds(base, _ROWS_PER_W)], xall_v)
    lane = lax.iota(jnp.int32, _L)

    def idx_body(j, carry):
        sl = pl.ds(j * _L, _L)
        xv = xall_v[sl]
        col = lax.rem(base + j * _L + lane, _NUM_FIELDS)
        xn = xv + col * _FIELD_DIM
        idxq_v[sl] = lax.div(xn, _QPR)
        idxr_v[sl] = lax.rem(xn, _BUCKET)
        return carry

    lax.fori_loop(0, _ROWS_PER_W // _L, idx_body, 0, unroll=4)

    def fire_gather(c, qc_v, rc_v, sem):
        iq = idxq_v.at[pl.ds(c * _CHUNK, _CHUNK)]
        ir = idxr_v.at[pl.ds(c * _CHUNK, _CHUNK)]
        pltpu.async_copy(qd_hbm.at[iq], qc_v, sem)
        pltpu.async_copy(rd_hbm.at[ir], rc_v, sem)

    def wait_gather(c, qc_v, rc_v, sem):
        iq = idxq_v.at[pl.ds(c * _CHUNK, _CHUNK)]
        ir = idxr_v.at[pl.ds(c * _CHUNK, _CHUNK)]
        pltpu.make_async_copy(qd_hbm.at[iq], qc_v, sem).wait()
        pltpu.make_async_copy(rd_hbm.at[ir], rc_v, sem).wait()

    def compute(qc_v, rc_v, o_v):
        def b_body(b, carry):
            def f_body(f, carry2):
                i = b * _NUM_FIELDS + f
                for k in range(_D // _L):
                    sl = pl.ds(k * _L, _L)
                    o_v[b, f, sl] = qc_v[i, sl] + rc_v[i, sl]
                return carry2

            lax.fori_loop(0, _NUM_FIELDS, f_body, 0, unroll=2)
            return carry

        lax.fori_loop(0, _NB, b_body, 0)

    def store_slice(c, o_v, sem):
        dst = out_hbm.at[pl.ds(bbase + c * _NB, _NB)]
        return pltpu.make_async_copy(o_v, dst, sem)

    fire_gather(0, qc_a, rc_a, sem_a)
    fire_gather(1, qc_b, rc_b, sem_b)

    def step(g, carry):
        c_a = 2 * g
        c_b = c_a + 1

        wait_gather(c_a, qc_a, rc_a, sem_a)
        compute(qc_a, rc_a, o_a)

        @pl.when(g > 0)
        def _():
            store_slice(c_a - 2, o_a, sem_oa).wait()

        store_slice(c_a, o_a, sem_oa).start()

        @pl.when(g < (_NCHUNKS // 2 - 1))
        def _():
            fire_gather(c_a + 2, qc_a, rc_a, sem_a)

        wait_gather(c_b, qc_b, rc_b, sem_b)
        compute(qc_b, rc_b, o_b)

        @pl.when(g > 0)
        def _():
            store_slice(c_b - 2, o_b, sem_ob).wait()

        store_slice(c_b, o_b, sem_ob).start()

        @pl.when(g < (_NCHUNKS // 2 - 1))
        def _():
            fire_gather(c_b + 2, qc_b, rc_b, sem_b)

        return carry

    lax.fori_loop(0, _NCHUNKS // 2, step, 0)

    store_slice(_NCHUNKS - 2, o_a, sem_oa).wait()
    store_slice(_NCHUNKS - 1, o_b, sem_ob).wait()


_mesh = plsc.VectorSubcoreMesh(core_axis_name="c", subcore_axis_name="s")

_ce_kernel = functools.partial(
    pl.kernel,
    out_type=jax.ShapeDtypeStruct((_B, _NUM_FIELDS, _D), jnp.float32),
    mesh=_mesh,
    scratch_types=[
        pltpu.VMEM((_ROWS_PER_W,), jnp.int32),
        pltpu.VMEM((_ROWS_PER_W,), jnp.int32),
        pltpu.VMEM((_ROWS_PER_W,), jnp.int32),
        pltpu.VMEM((_CHUNK, 2 * _D), jnp.float32),
        pltpu.VMEM((_CHUNK, 2 * _D), jnp.float32),
        pltpu.VMEM((_NB, _NUM_FIELDS, _D), jnp.float32),
        pltpu.VMEM((_CHUNK, 2 * _D), jnp.float32),
        pltpu.VMEM((_CHUNK, 2 * _D), jnp.float32),
        pltpu.VMEM((_NB, _NUM_FIELDS, _D), jnp.float32),
        pltpu.SemaphoreType.DMA,
        pltpu.SemaphoreType.DMA,
        pltpu.SemaphoreType.DMA,
        pltpu.SemaphoreType.DMA,
    ],
    compiler_params=pltpu.CompilerParams(use_tc_tiling_on_sc=True),
)(_sc_body)


def kernel(x, Q_v, R_v, Q_s, R_s):
    qd, rd = _prune_tables(Q_v, Q_s, R_v, R_s)
    x_flat = x.reshape(_N)
    return _ce_kernel(x_flat, qd, rd)
